# baseline (speedup 1.0000x reference)
import dataclasses
import functools

import jax
import jax.numpy as jnp
import numpy as np
from jax import lax
from jax.experimental import pallas as pl
from jax.experimental.pallas import tpu as pltpu
from jax.experimental.pallas import tpu_sc as plsc

_RES = 64
_NBINS = _RES * _RES
_B = 16
_N = 16384
_C = 128
_HALF = _N // 2
_HBINS = _NBINS // 2
_GCHUNK = 64


def _bin_body(pts_ref, r_ref, j_ref):
    x = pts_ref[:, 0, :]
    y = pts_ref[:, 1, :]
    z = pts_ref[:, 2, :]
    r = jnp.sqrt(x * x + y * y + z * z)
    t = np.pi / float(_RES)
    k = 2 * np.pi / float(_RES)
    inv_t = np.float32(1.0 / t)
    inv_k = np.float32(1.0 / k)
    u = jnp.clip(z / r, -1.0, 1.0)
    acos_u = jnp.arctan2(jnp.sqrt((1.0 - u) * (1.0 + u)), u)
    phi = jnp.round(acos_u * inv_t).astype(jnp.int32) % _RES
    th = jnp.arctan2(y, x)
    rho = jnp.where(y >= 0, jnp.round(th * inv_k),
                    jnp.round((th + 2 * np.pi) * inv_k))
    rho = rho.astype(jnp.int32) % _RES
    r_ref[...] = r
    j_ref[...] = rho * _RES + phi


def _sc_body(r_hbm, j_hbm, feat_hbm, rmin_hbm, gat_hbm,
             r_v, j_v, rep_v, rmin_v, widx_v, widx2_v, gbuf0, gbuf1, shf, shi,
             sem0, sem1):
    c = lax.axis_index("c")
    s = lax.axis_index("s")
    b = c * 8 + s // 2
    h = s % 2
    partner = s + 1 - 2 * h

    lane = lax.iota(jnp.int32, 16)
    lane_base = lane * _NBINS
    inf16 = jnp.full((16,), jnp.inf, jnp.float32)
    zero16 = jnp.zeros((16,), jnp.int32)

    pltpu.sync_copy(r_hbm.at[b, pl.ds(h * _HALF, _HALF)], r_v)
    pltpu.sync_copy(j_hbm.at[b, pl.ds(h * _HALF, _HALF)], j_v)

    @pl.loop(0, 16 * _NBINS, step=16)
    def _(i):
        rep_v[pl.ds(i, 16)] = inf16

    @pl.loop(0, _HALF, step=16)
    def _(i):
        jv = j_v[pl.ds(i, 16)]
        rv = r_v[pl.ds(i, 16)]
        addr = jv + lane_base
        cur = plsc.load_gather(rep_v, [addr])
        plsc.store_scatter(rep_v, [addr], jnp.minimum(cur, rv))

    @pl.loop(0, _NBINS, step=16)
    def _(v):
        acc = rep_v[pl.ds(v, 16)]
        for cpy in range(1, 16):
            acc = jnp.minimum(acc, rep_v[pl.ds(cpy * _NBINS + v, 16)])
        rmin_v[pl.ds(v, 16)] = acc

    pltpu.sync_copy(rmin_v, shf.at[s])
    plsc.subcore_barrier()
    pltpu.sync_copy(shf.at[partner], rep_v.at[pl.ds(0, _NBINS)])

    @pl.loop(0, _NBINS, step=16)
    def _(v):
        rmin_v[pl.ds(v, 16)] = jnp.minimum(rmin_v[pl.ds(v, 16)], rep_v[pl.ds(v, 16)])

    @pl.loop(0, _NBINS, step=16)
    def _(v):
        widx_v[pl.ds(v, 16)] = zero16

    @pl.loop(0, _HALF, step=16)
    def _(i):
        jv = j_v[pl.ds(i, 16)]
        rv = r_v[pl.ds(i, 16)]
        cur = plsc.load_gather(rmin_v, [jv])
        plsc.store_scatter(widx_v, [jv], lane + (h * _HALF + i), mask=rv == cur)

    pltpu.sync_copy(widx_v, shi.at[s])
    plsc.subcore_barrier()
    pltpu.sync_copy(shi.at[partner], widx2_v)

    @pl.loop(0, _NBINS, step=16)
    def _(v):
        widx_v[pl.ds(v, 16)] = jnp.maximum(widx_v[pl.ds(v, 16)], widx2_v[pl.ds(v, 16)])

    @pl.when(h == 0)
    def _():
        pltpu.sync_copy(rmin_v, rmin_hbm.at[b])

    bbase = h * _HBINS

    @pl.loop(0, _HBINS // _GCHUNK, step=2)
    def _(ci):
        idx0 = widx_v.at[pl.ds(bbase + ci * _GCHUNK, _GCHUNK)]
        idx1 = widx_v.at[pl.ds(bbase + (ci + 1) * _GCHUNK, _GCHUNK)]
        cp0 = pltpu.async_copy(feat_hbm.at[b].at[idx0], gbuf0, sem0)
        cp1 = pltpu.async_copy(feat_hbm.at[b].at[idx1], gbuf1, sem1)
        cp0.wait()
        pltpu.sync_copy(gbuf0, gat_hbm.at[b, pl.ds(bbase + ci * _GCHUNK, _GCHUNK)])
        cp1.wait()
        pltpu.sync_copy(gbuf1, gat_hbm.at[b, pl.ds(bbase + (ci + 1) * _GCHUNK, _GCHUNK)])


def _fin_body(g_ref, rmin_ref, feat_ref, dis_ref):
    g = g_ref[0]
    rm = rmin_ref[0]
    m = (rm < jnp.float32(1e30)).astype(jnp.float32)
    gt = jnp.transpose(g, (1, 0))
    feat_ref[...] = (gt * m).reshape(1, _C, _RES, _RES)
    dis_ref[...] = jnp.where(rm < jnp.float32(1e30), rm, 0.0).reshape(1, 1, _RES, _RES)


def kernel(pts, feat):
    pts_t = jnp.transpose(pts, (0, 2, 1))

    r, j = pl.pallas_call(
        _bin_body,
        in_specs=[pl.BlockSpec((_B, 3, _N), lambda: (0, 0, 0))],
        out_specs=[pl.BlockSpec((_B, _N), lambda: (0, 0)),
                   pl.BlockSpec((_B, _N), lambda: (0, 0))],
        out_shape=[jax.ShapeDtypeStruct((_B, _N), jnp.float32),
                   jax.ShapeDtypeStruct((_B, _N), jnp.int32)],
    )(pts_t)

    cp = pltpu.CompilerParams()
    if "needs_layout_passes" in pltpu.CompilerParams.__dataclass_fields__:
        cp = dataclasses.replace(cp, needs_layout_passes=False)
    sc_kernel = functools.partial(
        pl.kernel,
        out_type=[jax.ShapeDtypeStruct((_B, _NBINS), jnp.float32),
                  jax.ShapeDtypeStruct((_B, _NBINS, _C), jnp.float32)],
        compiler_params=cp,
        mesh=plsc.VectorSubcoreMesh(core_axis_name="c", subcore_axis_name="s"),
        scratch_types=[
            pltpu.VMEM((_HALF,), jnp.float32),
            pltpu.VMEM((_HALF,), jnp.int32),
            pltpu.VMEM((16 * _NBINS,), jnp.float32),
            pltpu.VMEM((_NBINS,), jnp.float32),
            pltpu.VMEM((_NBINS,), jnp.int32),
            pltpu.VMEM((_NBINS,), jnp.int32),
            pltpu.VMEM((_GCHUNK, _C), jnp.float32),
            pltpu.VMEM((_GCHUNK, _C), jnp.float32),
            pltpu.VMEM_SHARED((16, _NBINS), jnp.float32),
            pltpu.VMEM_SHARED((16, _NBINS), jnp.int32),
            pltpu.SemaphoreType.DMA,
            pltpu.SemaphoreType.DMA,
        ],
    )(_sc_body)
    rmin, gat = sc_kernel(r, j, feat)

    feat_out, dis_out = pl.pallas_call(
        _fin_body,
        grid=(_B,),
        in_specs=[pl.BlockSpec((1, _NBINS, _C), lambda b: (b, 0, 0)),
                  pl.BlockSpec((1, 1, _NBINS), lambda b: (b, 0, 0))],
        out_specs=[pl.BlockSpec((1, _C, _RES, _RES), lambda b: (b, 0, 0, 0)),
                   pl.BlockSpec((1, 1, _RES, _RES), lambda b: (b, 0, 0, 0))],
        out_shape=[jax.ShapeDtypeStruct((_B, _C, _RES, _RES), jnp.float32),
                   jax.ShapeDtypeStruct((_B, 1, _RES, _RES), jnp.float32)],
    )(gat, rmin.reshape(_B, 1, _NBINS))

    return dis_out, feat_out


# speedup vs baseline: 1.1726x; 1.1726x over previous
import dataclasses
import functools

import jax
import jax.numpy as jnp
import numpy as np
from jax import lax
from jax.experimental import pallas as pl
from jax.experimental.pallas import tpu as pltpu
from jax.experimental.pallas import tpu_sc as plsc

_RES = 64
_NBINS = _RES * _RES
_B = 16
_N = 16384
_C = 128
_HALF = _N // 2
_HBINS = _NBINS // 2
_GCHUNK = 64


def _bin_body(pts_ref, r_ref, j_ref):
    x = pts_ref[:, 0, :]
    y = pts_ref[:, 1, :]
    z = pts_ref[:, 2, :]
    r = jnp.sqrt(x * x + y * y + z * z)
    t = np.pi / float(_RES)
    k = 2 * np.pi / float(_RES)
    inv_t = np.float32(1.0 / t)
    inv_k = np.float32(1.0 / k)
    u = jnp.clip(z / r, -1.0, 1.0)
    acos_u = jnp.arctan2(jnp.sqrt((1.0 - u) * (1.0 + u)), u)
    phi = jnp.round(acos_u * inv_t).astype(jnp.int32) % _RES
    th = jnp.arctan2(y, x)
    rho = jnp.where(y >= 0, jnp.round(th * inv_k),
                    jnp.round((th + 2 * np.pi) * inv_k))
    rho = rho.astype(jnp.int32) % _RES
    r_ref[...] = r
    j_ref[...] = rho * _RES + phi


def _sc_body(r_hbm, j_hbm, feat_hbm, rmin_hbm, gat_hbm,
             r_v, j_v, rep_v, rmin_v, widx_v, widx2_v, gbuf0, gbuf1, shf, shi,
             sem0, sem1):
    c = lax.axis_index("c")
    s = lax.axis_index("s")
    b = c * 8 + s // 2
    h = s % 2
    partner = s + 1 - 2 * h

    lane = lax.iota(jnp.int32, 16)
    lane_base = lane * _NBINS
    inf16 = jnp.full((16,), jnp.inf, jnp.float32)
    zero16 = jnp.zeros((16,), jnp.int32)

    pltpu.sync_copy(r_hbm.at[b, pl.ds(h * _HALF, _HALF)], r_v)
    pltpu.sync_copy(j_hbm.at[b, pl.ds(h * _HALF, _HALF)], j_v)

    @pl.loop(0, 16 * _NBINS, step=16)
    def _(i):
        rep_v[pl.ds(i, 16)] = inf16

    @pl.loop(0, _HALF, step=16)
    def _(i):
        jv = j_v[pl.ds(i, 16)]
        rv = r_v[pl.ds(i, 16)]
        addr = jv + lane_base
        cur = plsc.load_gather(rep_v, [addr])
        plsc.store_scatter(rep_v, [addr], jnp.minimum(cur, rv))

    @pl.loop(0, _NBINS, step=16)
    def _(v):
        acc = rep_v[pl.ds(v, 16)]
        for cpy in range(1, 16):
            acc = jnp.minimum(acc, rep_v[pl.ds(cpy * _NBINS + v, 16)])
        rmin_v[pl.ds(v, 16)] = acc

    pltpu.sync_copy(rmin_v, shf.at[s])
    plsc.subcore_barrier()
    pltpu.sync_copy(shf.at[partner], rep_v.at[pl.ds(0, _NBINS)])

    @pl.loop(0, _NBINS, step=16)
    def _(v):
        rmin_v[pl.ds(v, 16)] = jnp.minimum(rmin_v[pl.ds(v, 16)], rep_v[pl.ds(v, 16)])

    @pl.loop(0, _NBINS, step=16)
    def _(v):
        widx_v[pl.ds(v, 16)] = zero16

    @pl.loop(0, _HALF, step=16)
    def _(i):
        jv = j_v[pl.ds(i, 16)]
        rv = r_v[pl.ds(i, 16)]
        cur = plsc.load_gather(rmin_v, [jv])
        plsc.store_scatter(widx_v, [jv], lane + (h * _HALF + i), mask=rv == cur)

    pltpu.sync_copy(widx_v, shi.at[s])
    plsc.subcore_barrier()
    pltpu.sync_copy(shi.at[partner], widx2_v)

    @pl.loop(0, _NBINS, step=16)
    def _(v):
        widx_v[pl.ds(v, 16)] = jnp.maximum(widx_v[pl.ds(v, 16)], widx2_v[pl.ds(v, 16)])

    @pl.when(h == 0)
    def _():
        pltpu.sync_copy(rmin_v, rmin_hbm.at[b])

    bbase = h * _HBINS

    @pl.loop(0, _HBINS // _GCHUNK, step=2)
    def _(ci):
        idx0 = widx_v.at[pl.ds(bbase + ci * _GCHUNK, _GCHUNK)]
        idx1 = widx_v.at[pl.ds(bbase + (ci + 1) * _GCHUNK, _GCHUNK)]
        cp0 = pltpu.async_copy(feat_hbm.at[b].at[idx0], gbuf0, sem0)
        cp1 = pltpu.async_copy(feat_hbm.at[b].at[idx1], gbuf1, sem1)
        cp0.wait()
        pltpu.sync_copy(gbuf0, gat_hbm.at[b, pl.ds(bbase + ci * _GCHUNK, _GCHUNK)])
        cp1.wait()
        pltpu.sync_copy(gbuf1, gat_hbm.at[b, pl.ds(bbase + (ci + 1) * _GCHUNK, _GCHUNK)])


def _fin_body(g_ref, rmin_ref, feat_ref, dis_ref):
    g = g_ref[0]
    rm = rmin_ref[0]
    m = (rm < jnp.float32(1e30)).astype(jnp.float32)
    gt = jnp.transpose(g, (1, 0))
    feat_ref[...] = gt[None] * m[:, None, :]
    dis_ref[...] = jnp.where(rm < jnp.float32(1e30), rm, 0.0)[:, None, :]


def kernel(pts, feat):
    pts_t = jnp.transpose(pts, (0, 2, 1))

    r, j = pl.pallas_call(
        _bin_body,
        in_specs=[pl.BlockSpec((_B, 3, _N), lambda: (0, 0, 0))],
        out_specs=[pl.BlockSpec((_B, _N), lambda: (0, 0)),
                   pl.BlockSpec((_B, _N), lambda: (0, 0))],
        out_shape=[jax.ShapeDtypeStruct((_B, _N), jnp.float32),
                   jax.ShapeDtypeStruct((_B, _N), jnp.int32)],
    )(pts_t)

    cp = pltpu.CompilerParams()
    if "needs_layout_passes" in pltpu.CompilerParams.__dataclass_fields__:
        cp = dataclasses.replace(cp, needs_layout_passes=False)
    sc_kernel = functools.partial(
        pl.kernel,
        out_type=[jax.ShapeDtypeStruct((_B, _NBINS), jnp.float32),
                  jax.ShapeDtypeStruct((_B, _NBINS, _C), jnp.float32)],
        compiler_params=cp,
        mesh=plsc.VectorSubcoreMesh(core_axis_name="c", subcore_axis_name="s"),
        scratch_types=[
            pltpu.VMEM((_HALF,), jnp.float32),
            pltpu.VMEM((_HALF,), jnp.int32),
            pltpu.VMEM((16 * _NBINS,), jnp.float32),
            pltpu.VMEM((_NBINS,), jnp.float32),
            pltpu.VMEM((_NBINS,), jnp.int32),
            pltpu.VMEM((_NBINS,), jnp.int32),
            pltpu.VMEM((_GCHUNK, _C), jnp.float32),
            pltpu.VMEM((_GCHUNK, _C), jnp.float32),
            pltpu.VMEM_SHARED((16, _NBINS), jnp.float32),
            pltpu.VMEM_SHARED((16, _NBINS), jnp.int32),
            pltpu.SemaphoreType.DMA,
            pltpu.SemaphoreType.DMA,
        ],
    )(_sc_body)
    rmin, gat = sc_kernel(r, j, feat)

    feat_t, dis_t = pl.pallas_call(
        _fin_body,
        grid=(_B,),
        in_specs=[pl.BlockSpec((1, _NBINS, _C), lambda b: (b, 0, 0)),
                  pl.BlockSpec((1, 1, _NBINS), lambda b: (b, 0, 0))],
        out_specs=[pl.BlockSpec((1, _C, _NBINS), lambda b: (b, 0, 0)),
                   pl.BlockSpec((1, 1, _NBINS), lambda b: (b, 0, 0))],
        out_shape=[jax.ShapeDtypeStruct((_B, _C, _NBINS), jnp.float32),
                   jax.ShapeDtypeStruct((_B, 1, _NBINS), jnp.float32)],
    )(gat, rmin.reshape(_B, 1, _NBINS))

    dis_out = dis_t.reshape(_B, 1, _RES, _RES)
    feat_out = feat_t.reshape(_B, _C, _RES, _RES)
    return dis_out, feat_out
